# speedup vs baseline: 1.3224x; 1.3224x over previous
import functools

import jax
import jax.numpy as jnp
from jax import lax
from jax.experimental import pallas as pl
from jax.experimental.pallas import tpu as pltpu
from jax.experimental.pallas import tpu_sc as plsc

VOCAB = 30523
D = 768
B = 4096 * 50
NC = 2
NS = 16
NW = NC * NS
BPW = B // NW
K = 128
NCHUNK = BPW // K

_mesh = plsc.VectorSubcoreMesh(core_axis_name="c", subcore_axis_name="s")


@functools.partial(
    pl.kernel,
    mesh=_mesh,
    out_type=jax.ShapeDtypeStruct((B, D), jnp.float32),
    scratch_types=[
        pltpu.VMEM((BPW,), jnp.int32),
        pltpu.VMEM((K, D), jnp.float32),
        pltpu.SemaphoreType.DMA,
    ],
)
def _emb_lookup(x_hbm, table_hbm, out_hbm, idx_v, rows_v, sem):
    wid = lax.axis_index("s") * NC + lax.axis_index("c")
    base = wid * BPW
    pltpu.sync_copy(x_hbm.at[pl.ds(base, BPW)], idx_v)

    def chunk(g, carry):
        off = g * K
        pltpu.async_copy(
            table_hbm.at[idx_v.at[pl.ds(off, K)]], rows_v, sem
        ).wait()
        pltpu.sync_copy(rows_v, out_hbm.at[pl.ds(base + off, K)])
        return carry

    lax.fori_loop(0, NCHUNK, chunk, 0, unroll=False)


def kernel(x, emb_weight):
    out = _emb_lookup(x.reshape(B).astype(jnp.int32), emb_weight)
    return out.reshape(4096, 50, D)


# speedup vs baseline: 1.3356x; 1.0100x over previous
import functools

import jax
import jax.numpy as jnp
from jax import lax
from jax.experimental import pallas as pl
from jax.experimental.pallas import tpu as pltpu
from jax.experimental.pallas import tpu_sc as plsc

VOCAB = 30523
D = 768
B = 4096 * 50
NC = 2
NS = 16
NW = NC * NS
BPW = B // NW
K = 80
NCHUNK = BPW // K
NBUF = 2

_mesh = plsc.VectorSubcoreMesh(core_axis_name="c", subcore_axis_name="s")


@functools.partial(
    pl.kernel,
    mesh=_mesh,
    out_type=jax.ShapeDtypeStruct((B, D), jnp.float32),
    scratch_types=[
        pltpu.VMEM((BPW,), jnp.int32),
        pltpu.VMEM((NBUF, K, D), jnp.float32),
        pltpu.SemaphoreType.DMA,
        pltpu.SemaphoreType.DMA,
        pltpu.SemaphoreType.DMA,
        pltpu.SemaphoreType.DMA,
    ],
)
def _emb_lookup(x_hbm, table_hbm, out_hbm, idx_v, rows_v, g0, g1, s0, s1):
    gsem = (g0, g1)
    ssem = (s0, s1)
    wid = lax.axis_index("s") * NC + lax.axis_index("c")
    base = wid * BPW
    pltpu.sync_copy(x_hbm.at[pl.ds(base, BPW)], idx_v)

    def gather_desc(g, b):
        return pltpu.make_async_copy(
            table_hbm.at[idx_v.at[pl.ds(g * K, K)]], rows_v.at[b], gsem[b]
        )

    def scatter_desc(g, b):
        return pltpu.make_async_copy(
            rows_v.at[b], out_hbm.at[pl.ds(base + g * K, K)], ssem[b]
        )

    for b in range(NBUF):
        gather_desc(b, b).start()

    def body(i, carry):
        g2 = i * NBUF
        for b in range(NBUF):
            g = g2 + b
            gather_desc(g, b).wait()
            scatter_desc(g, b).start()

            @pl.when(g + NBUF < NCHUNK)
            def _():
                scatter_desc(g, b).wait()
                gather_desc(g + NBUF, b).start()

        return carry

    lax.fori_loop(0, NCHUNK // NBUF, body, 0, unroll=False)

    for b in range(NBUF):
        scatter_desc(NCHUNK - NBUF + b, b).wait()


def kernel(x, emb_weight):
    out = _emb_lookup(x.reshape(B).astype(jnp.int32), emb_weight)
    return out.reshape(4096, 50, D)
